# Initial kernel scaffold; baseline (speedup 1.0000x reference)
import jax
import jax.numpy as jnp
from jax import lax
from jax.experimental import pallas as pl
from jax.experimental.pallas import tpu as pltpu
from jax.experimental.pallas import tpu_sc as plsc

_VOCAB = 1000000
_FEATURES = 32
_BATCH = 16384
_HIST = 50

_NC = 2
_NS = 16
_NW = _NC * _NS
_BPW = _BATCH // _NW
_NIDX = _BPW * _HIST
_CHUNK = 128
_NCHUNK = _NIDX // _CHUNK
_INFLIGHT = 8
_LANES = 16


def _sc_body(idx_hbm, tab_hbm, out_hbm, idx_v, vals_v, out_v, sem):
  wid = lax.axis_index("s") * _NC + lax.axis_index("c")
  base = wid * _NIDX

  pltpu.sync_copy(idx_hbm.at[pl.ds(base, _NIDX)], idx_v)

  @pl.loop(0, _NIDX // _LANES, unroll=8)
  def _scale(i):
    sl = pl.ds(i * _LANES, _LANES)
    idx_v[sl] = idx_v[sl] * _FEATURES

  @pl.loop(0, _NCHUNK // _INFLIGHT)
  def _gather(j):
    descs = []
    for b in range(_INFLIGHT):
      off = (j * _INFLIGHT + b) * _CHUNK
      descs.append(
          pltpu.async_copy(
              tab_hbm.at[idx_v.at[pl.ds(off, _CHUNK)]],
              vals_v.at[pl.ds(off, _CHUNK)],
              sem,
          )
      )
    for d in descs:
      d.wait()

  iota = lax.iota(jnp.int32, _LANES)

  @pl.loop(0, _BPW // _LANES)
  def _reduce(g):
    acc = jnp.zeros((_LANES,), jnp.float32)
    gbase = g * (_LANES * _HIST)
    for l in range(_HIST):
      iv = gbase + l + iota * _HIST
      acc = acc + plsc.load_gather(vals_v, [iv])
    out_v[pl.ds(g * _LANES, _LANES)] = acc * (1.0 / _HIST)

  pltpu.sync_copy(out_v, out_hbm.at[pl.ds(wid * _BPW, _BPW)])


@jax.jit
def _sc_embedding_mean(idx_flat, tab_flat):
  mesh = plsc.VectorSubcoreMesh(core_axis_name="c", subcore_axis_name="s")
  return pl.kernel(
      _sc_body,
      out_type=jax.ShapeDtypeStruct((_BATCH,), jnp.float32),
      mesh=mesh,
      scratch_types=[
          pltpu.VMEM((_NIDX,), jnp.int32),
          pltpu.VMEM((_NIDX,), jnp.float32),
          pltpu.VMEM((_BPW,), jnp.float32),
          pltpu.SemaphoreType.DMA,
      ],
  )(idx_flat, tab_flat)


def kernel(index, table):
  idx_flat = index.reshape(_BATCH * _HIST).astype(jnp.int32)
  tab_flat = table.reshape(_VOCAB * _FEATURES)
  out = _sc_embedding_mean(idx_flat, tab_flat)
  return out.reshape(_BATCH, 1)


# speedup vs baseline: 2.8469x; 2.8469x over previous
import jax
import jax.numpy as jnp
from jax import lax
from jax.experimental import pallas as pl
from jax.experimental.pallas import tpu as pltpu
from jax.experimental.pallas import tpu_sc as plsc

_VOCAB = 1000000
_FEATURES = 32
_BATCH = 16384
_HIST = 50

_NC = 2
_NS = 16
_NW = _NC * _NS
_BPW = _BATCH // _NW
_NIDX = _BPW * _HIST
_CHUNK = 128
_NCHUNK = _NIDX // _CHUNK
_INFLIGHT = 8
_LANES = 16


def _sc_body(idx_hbm, tab_hbm, out_hbm, idx_v, vals_v, out_v, sem):
  wid = lax.axis_index("s") * _NC + lax.axis_index("c")
  base = wid * _NIDX

  pltpu.sync_copy(idx_hbm.at[pl.ds(base, _NIDX)], idx_v)

  @pl.loop(0, _NIDX // _LANES, unroll=8)
  def _scale(i):
    sl = pl.ds(i * _LANES, _LANES)
    idx_v[sl] = idx_v[sl] * _FEATURES

  @pl.loop(0, _NCHUNK // _INFLIGHT)
  def _gather(j):
    descs = []
    for b in range(_INFLIGHT):
      off = (j * _INFLIGHT + b) * _CHUNK
      descs.append(
          pltpu.async_copy(
              tab_hbm.at[idx_v.at[pl.ds(off, _CHUNK)]],
              vals_v.at[pl.ds(off, _CHUNK)],
              sem,
          )
      )
    for d in descs:
      d.wait()

  iota = lax.iota(jnp.int32, _LANES)

  @pl.loop(0, _BPW // _LANES)
  def _reduce(g):
    acc = jnp.zeros((_LANES,), jnp.float32)
    gbase = g * (_LANES * _HIST)
    for l in range(_HIST):
      iv = gbase + l + iota * _HIST
      acc = acc + plsc.load_gather(vals_v, [iv])
    out_v[pl.ds(g * _LANES, _LANES)] = acc * (1.0 / _HIST)

  pltpu.sync_copy(out_v, out_hbm.at[pl.ds(wid * _BPW, _BPW)])


@jax.jit
def _sc_embedding_mean(idx_flat, tab_flat):
  mesh = plsc.VectorSubcoreMesh(core_axis_name="c", subcore_axis_name="s")
  return pl.kernel(
      _sc_body,
      out_type=jax.ShapeDtypeStruct((_BATCH,), jnp.float32),
      mesh=mesh,
      compiler_params=pltpu.CompilerParams(needs_layout_passes=False),
      scratch_types=[
          pltpu.VMEM((_NIDX,), jnp.int32),
          pltpu.VMEM((_NIDX,), jnp.float32),
          pltpu.VMEM((_BPW,), jnp.float32),
          pltpu.SemaphoreType.DMA,
      ],
  )(idx_flat, tab_flat)


def kernel(index, table):
  idx_flat = index.reshape(_BATCH * _HIST).astype(jnp.int32)
  tab_flat = table.reshape(_VOCAB * _FEATURES)
  out = _sc_embedding_mean(idx_flat, tab_flat)
  return out.reshape(_BATCH, 1)
